# baseline (speedup 1.0000x reference)
import jax
import jax.numpy as jnp
from jax.experimental import pallas as pl

DEPTH = 1000
N0 = 4096
N1 = 26
RB = 200


def _body(idx_ref, out_ref):
    i = pl.program_id(0)
    row = idx_ref[0]
    iota = jax.lax.broadcasted_iota(jnp.int32, (RB, N0), 0) + (i % 5) * RB
    out_ref[...] = (iota == row).astype(jnp.float32)


def kernel(inputs):
    idx_t = inputs.astype(jnp.int32).T.reshape(N1, 1, N0)
    out_t = pl.pallas_call(
        _body,
        grid=(N1 * DEPTH // RB,),
        in_specs=[pl.BlockSpec((1, 1, N0), lambda i: (i // 5, 0, 0))],
        out_specs=pl.BlockSpec((RB, N0), lambda i: (i, 0)),
        out_shape=jax.ShapeDtypeStruct((N1 * DEPTH, N0), jnp.float32),
    )(idx_t)
    return out_t.reshape(N1, DEPTH, N0).transpose(2, 0, 1)


# speedup vs baseline: 1.0200x; 1.0200x over previous
import jax
import jax.numpy as jnp
from jax.experimental import pallas as pl

DEPTH = 1000
N0 = 4096
N1 = 26


def _body(idx_ref, out_ref):
    row = idx_ref[0]
    iota = jax.lax.broadcasted_iota(jnp.int32, (DEPTH, N0), 0)
    out_ref[...] = (iota == row).astype(jnp.float32)


def kernel(inputs):
    idx_t = inputs.astype(jnp.int32).T.reshape(N1, 1, N0)
    out_t = pl.pallas_call(
        _body,
        grid=(N1,),
        in_specs=[pl.BlockSpec((1, 1, N0), lambda i: (i, 0, 0))],
        out_specs=pl.BlockSpec((DEPTH, N0), lambda i: (i, 0)),
        out_shape=jax.ShapeDtypeStruct((N1 * DEPTH, N0), jnp.float32),
    )(idx_t)
    return out_t.reshape(N1, DEPTH, N0).transpose(2, 0, 1)
